# Initial kernel scaffold; baseline (speedup 1.0000x reference)
import jax
import jax.numpy as jnp
from jax.experimental import pallas as pl


def kernel(x, edge_index, W1, a_src1, a_dst1, b1, W2, a_src2, a_dst2, b2, Wp, bp):
    raise NotImplementedError("write your pallas kernel here")


# speedup vs baseline: 1.0037x; 1.0037x over previous
import jax
import jax.numpy as jnp
from jax.experimental import pallas as pl


def _mm_kernel(x_ref, w_ref, o_ref):
    o_ref[...] = jnp.dot(x_ref[...], w_ref[...], preferred_element_type=jnp.float32)


def _matmul(x, w):
    M, K = x.shape
    _, Hd = w.shape
    BM = 1000
    return pl.pallas_call(
        _mm_kernel,
        grid=(M // BM,),
        in_specs=[
            pl.BlockSpec((BM, K), lambda i: (i, 0)),
            pl.BlockSpec((K, Hd), lambda i: (0, 0)),
        ],
        out_specs=pl.BlockSpec((BM, Hd), lambda i: (i, 0)),
        out_shape=jax.ShapeDtypeStruct((M, Hd), jnp.float32),
    )(x, w)


def _gat_layer_v0(x, src, dst, W, a_src, a_dst, b):
    n = x.shape[0]
    xl = _matmul(x, W)
    loop = jnp.arange(n, dtype=src.dtype)
    s = jnp.concatenate([src, loop])
    d = jnp.concatenate([dst, loop])
    alpha_s = (xl * a_src).sum(-1)
    alpha_d = (xl * a_dst).sum(-1)
    e = alpha_s[s] + alpha_d[d]
    e = jnp.where(e > 0, e, 0.2 * e)
    emax = jax.ops.segment_max(e, d, num_segments=n)
    emax = jnp.where(jnp.isfinite(emax), emax, 0.0)
    ex = jnp.exp(e - emax[d])
    denom = jax.ops.segment_sum(ex, d, num_segments=n)
    alpha = ex / (denom[d] + 1e-16)
    out = jax.ops.segment_sum(xl[s] * alpha[:, None], d, num_segments=n)
    return out + b


def kernel(x, edge_index, W1, a_src1, a_dst1, b1, W2, a_src2, a_dst2, b2, Wp, bp):
    src = edge_index[0]
    dst = edge_index[1]
    h = _gat_layer_v0(x, src, dst, W1, a_src1, a_dst1, b1)
    h = jax.nn.elu(h)
    h = _gat_layer_v0(h, src, dst, W2, a_src2, a_dst2, b2)
    h = _matmul(h, Wp) + bp
    h = jax.nn.relu(h)
    return h


# speedup vs baseline: 12.5179x; 12.4722x over previous
import functools

import jax
import jax.numpy as jnp
from jax import lax
from jax.experimental import pallas as pl
from jax.experimental.pallas import tpu as pltpu
from jax.experimental.pallas import tpu_sc as plsc

N = 10000
E = 640000
D_IN = 768
H = 128

NT = 32
RPT = 313
NPAD = NT * RPT
TPAD = NPAD + 320
ACCR = 320
K = 128
CH = 2000
FL = 2048
SS = 4096
LCAP = E + FL + K

_f32 = jnp.float32
_i32 = jnp.int32


def _proj_tail(xl, asv, adv, asr_ref, adr_ref, m_ref, first):
    dn = (((1,), (1,)), ((), ()))
    asr = lax.dot_general(asv, xl, dn, preferred_element_type=_f32)
    adr = lax.dot_general(adv, xl, dn, preferred_element_type=_f32)
    asr_ref[...] = asr[None]
    adr_ref[...] = adr[None]
    es = asr + adr
    es = jnp.where(es > 0.0, es, 0.2 * es)
    mb = jnp.max(es)

    @pl.when(first)
    def _():
        m_ref[...] = jnp.full((1, 1), -jnp.inf, _f32)

    m_ref[...] = jnp.maximum(m_ref[...], mb)


def _tc1_body(x_ref, w_ref, asv_ref, adv_ref, xl_ref, asr_ref, adr_ref, m_ref):
    xl = jnp.dot(x_ref[...], w_ref[...], preferred_element_type=_f32)
    xl_ref[...] = xl
    _proj_tail(xl, asv_ref[...], adv_ref[...], asr_ref, adr_ref, m_ref,
               pl.program_id(0) == 0)


def _tc1(x, W, a_src, a_dst):
    BM = 1000
    G = N // BM
    return pl.pallas_call(
        _tc1_body,
        grid=(G,),
        in_specs=[
            pl.BlockSpec((BM, x.shape[1]), lambda i: (i, 0)),
            pl.BlockSpec((x.shape[1], H), lambda i: (0, 0)),
            pl.BlockSpec((1, H), lambda i: (0, 0)),
            pl.BlockSpec((1, H), lambda i: (0, 0)),
        ],
        out_specs=[
            pl.BlockSpec((BM, H), lambda i: (i, 0)),
            pl.BlockSpec((1, 1, BM), lambda i: (i, 0, 0)),
            pl.BlockSpec((1, 1, BM), lambda i: (i, 0, 0)),
            pl.BlockSpec((1, 1), lambda i: (0, 0)),
        ],
        out_shape=[
            jax.ShapeDtypeStruct((N, H), _f32),
            jax.ShapeDtypeStruct((G, 1, BM), _f32),
            jax.ShapeDtypeStruct((G, 1, BM), _f32),
            jax.ShapeDtypeStruct((1, 1), _f32),
        ],
    )(x, W, a_src.reshape(1, H), a_dst.reshape(1, H))


def _norm_h(acc, den, xl, asv, adv, m, b):
    asc = jnp.sum(xl * asv, axis=1, keepdims=True)
    adc = jnp.sum(xl * adv, axis=1, keepdims=True)
    es = asc + adc
    es = jnp.where(es > 0.0, es, 0.2 * es)
    exs = jnp.exp(es - m)
    return (acc + exs * xl) / (den + exs + 1e-16) + b


def _tc2_body(acc_ref, den_ref, xl_ref, as1_ref, ad1_ref, m1_ref, b1_ref,
              w_ref, asv_ref, adv_ref, xl2_ref, asr_ref, adr_ref, m2_ref):
    h = _norm_h(acc_ref[...], den_ref[...], xl_ref[...], as1_ref[...],
                ad1_ref[...], m1_ref[0, 0], b1_ref[...])
    h = jnp.where(h > 0.0, h, jnp.exp(jnp.minimum(h, 0.0)) - 1.0)
    xl2 = jnp.dot(h, w_ref[...], preferred_element_type=_f32)
    xl2_ref[...] = xl2
    _proj_tail(xl2, asv_ref[...], adv_ref[...], asr_ref, adr_ref, m2_ref,
               pl.program_id(0) == 0)


def _tc2(acc, den_b, xl1, a_src1, a_dst1, m1, b1, W2, a_src2, a_dst2):
    BM = 1000
    G = N // BM
    vec = pl.BlockSpec((1, H), lambda i: (0, 0))
    blk = pl.BlockSpec((BM, H), lambda i: (i, 0))
    return pl.pallas_call(
        _tc2_body,
        grid=(G,),
        in_specs=[blk, blk, blk, vec, vec,
                  pl.BlockSpec((1, 1), lambda i: (0, 0)), vec,
                  pl.BlockSpec((H, H), lambda i: (0, 0)), vec, vec],
        out_specs=[
            pl.BlockSpec((BM, H), lambda i: (i, 0)),
            pl.BlockSpec((1, 1, BM), lambda i: (i, 0, 0)),
            pl.BlockSpec((1, 1, BM), lambda i: (i, 0, 0)),
            pl.BlockSpec((1, 1), lambda i: (0, 0)),
        ],
        out_shape=[
            jax.ShapeDtypeStruct((N, H), _f32),
            jax.ShapeDtypeStruct((G, 1, BM), _f32),
            jax.ShapeDtypeStruct((G, 1, BM), _f32),
            jax.ShapeDtypeStruct((1, 1), _f32),
        ],
    )(acc, den_b, xl1, a_src1.reshape(1, H), a_dst1.reshape(1, H), m1,
      b1.reshape(1, H), W2, a_src2.reshape(1, H), a_dst2.reshape(1, H))


def _tc3_body(acc_ref, den_ref, xl_ref, as2_ref, ad2_ref, m2_ref, b2_ref,
              wp_ref, bp_ref, out_ref):
    h = _norm_h(acc_ref[...], den_ref[...], xl_ref[...], as2_ref[...],
                ad2_ref[...], m2_ref[0, 0], b2_ref[...])
    o = jnp.dot(h, wp_ref[...], preferred_element_type=_f32) + bp_ref[...]
    out_ref[...] = jnp.maximum(o, 0.0)


def _tc3(acc, den_b, xl2, a_src2, a_dst2, m2, b2, Wp, bp):
    BM = 1000
    G = N // BM
    vec = pl.BlockSpec((1, H), lambda i: (0, 0))
    blk = pl.BlockSpec((BM, H), lambda i: (i, 0))
    return pl.pallas_call(
        _tc3_body,
        grid=(G,),
        in_specs=[blk, blk, blk, vec, vec,
                  pl.BlockSpec((1, 1), lambda i: (0, 0)), vec,
                  pl.BlockSpec((H, H), lambda i: (0, 0)), vec],
        out_specs=pl.BlockSpec((BM, H), lambda i: (i, 0)),
        out_shape=jax.ShapeDtypeStruct((N, H), _f32),
    )(acc, den_b, xl2, a_src2.reshape(1, H), a_dst2.reshape(1, H), m2,
      b2.reshape(1, H), Wp, bp.reshape(1, H))


def _wid():
    return lax.axis_index("s") * 2 + lax.axis_index("c")


def _filter_body(src_ref, dst_ref, slist_ref, dlist_ref, counts_ref,
                 sbuf, dbuf, stg_s, stg_d, cbuf):
    wid = _wid()
    lo = wid * RPT
    iot = lax.iota(_i32, 16)
    zero16 = jnp.zeros((16,), _i32)
    sent = jnp.full((16,), RPT, _i32)

    def _flush(cl, wt):
        fo = pl.multiple_of(wid * LCAP + wt, 8)
        pltpu.sync_copy(stg_s.at[pl.ds(0, FL)],
                        slist_ref.at[pl.ds(fo, FL)])
        pltpu.sync_copy(stg_d.at[pl.ds(0, FL)],
                        dlist_ref.at[pl.ds(fo, FL)])
        for t in range(128):
            sv = plsc.load_gather(stg_s, [FL + t * 16 + iot])
            dv = plsc.load_gather(stg_d, [FL + t * 16 + iot])
            plsc.store_scatter(stg_s, [t * 16 + iot], sv)
            plsc.store_scatter(stg_d, [t * 16 + iot], dv)
        return cl - FL, wt + FL

    def _noflush(cl, wt):
        return cl, wt

    def _chunk(ci, carry):
        cl, wt = carry
        off = pl.multiple_of(ci * CH, 8)
        pltpu.sync_copy(src_ref.at[pl.ds(off, CH)], sbuf)
        pltpu.sync_copy(dst_ref.at[pl.ds(off, CH)], dbuf)
        clv = jnp.full((16,), cl, _i32)
        for t in range(CH // 16):
            sv = sbuf[pl.ds(t * 16, 16)]
            dv = dbuf[pl.ds(t * 16, 16)]
            dloc = dv - lo
            msk = (dloc >= 0) & (dloc < RPT)
            pos = clv + plsc.cumsum(msk.astype(_i32)) - 1
            plsc.store_scatter(stg_s, [pos], sv, mask=msk)
            plsc.store_scatter(stg_d, [pos], dloc, mask=msk)
            clv = clv + plsc.all_reduce_population_count(msk)
        cl = jnp.max(clv)
        return lax.cond(cl >= FL, _flush, _noflush, cl, wt)

    cl, wt = lax.fori_loop(0, E // CH, _chunk, (jnp.int32(0), jnp.int32(0)))
    tcount = wt + cl
    for t in range(K // 16):
        pos = cl + t * 16 + iot
        plsc.store_scatter(stg_s, [pos], zero16)
        plsc.store_scatter(stg_d, [pos], sent)
    cl = cl + K
    cl, wt = lax.cond(cl >= FL, _flush, _noflush, cl, wt)
    fo = pl.multiple_of(wid * LCAP + wt, 8)
    pltpu.sync_copy(stg_s.at[pl.ds(0, FL)],
                    slist_ref.at[pl.ds(fo, FL)])
    pltpu.sync_copy(stg_d.at[pl.ds(0, FL)],
                    dlist_ref.at[pl.ds(fo, FL)])
    cbuf[...] = jnp.where(iot == 0, jnp.full((16,), tcount, _i32), 0)
    pltpu.sync_copy(cbuf, counts_ref.at[pl.ds(pl.multiple_of(wid * 16, 8), 16)])


@functools.partial(
    pl.kernel,
    out_type=[
        jax.ShapeDtypeStruct((NT * LCAP,), _i32),
        jax.ShapeDtypeStruct((NT * LCAP,), _i32),
        jax.ShapeDtypeStruct((NT * 16,), _i32),
    ],
    mesh=plsc.VectorSubcoreMesh(core_axis_name="c", subcore_axis_name="s"),
    compiler_params=pltpu.CompilerParams(needs_layout_passes=False),
    scratch_types=[
        pltpu.VMEM((CH,), _i32),
        pltpu.VMEM((CH,), _i32),
        pltpu.VMEM((SS,), _i32),
        pltpu.VMEM((SS,), _i32),
        pltpu.VMEM((16,), _i32),
    ],
)
def _filter(src_ref, dst_ref, slist_ref, dlist_ref, counts_ref,
            sbuf, dbuf, stg_s, stg_d, cbuf):
    _filter_body(src_ref, dst_ref, slist_ref, dlist_ref, counts_ref,
                 sbuf, dbuf, stg_s, stg_d, cbuf)


def _edge_body(slist_ref, dlist_ref, counts_ref, asq_ref, adq_ref, xl_ref,
               mv_ref, acc_ref, dens_ref,
               as_t, ad_t, mbuf, cbuf, slb, dlb, exb, rows, accv, denv, sem):
    wid = _wid()
    lo = wid * RPT
    iot = lax.iota(_i32, 16)
    lane0 = iot == 0
    zero16f = jnp.zeros((16,), _f32)

    pltpu.sync_copy(asq_ref, as_t)
    pltpu.sync_copy(adq_ref, ad_t)
    pltpu.sync_copy(mv_ref, mbuf)
    pltpu.sync_copy(counts_ref.at[pl.ds(pl.multiple_of(wid * 16, 8), 16)], cbuf)

    def _zero(i, _):
        plsc.store_scatter(accv, [i * 16 + iot], zero16f)
        return 0

    lax.fori_loop(0, ACCR * H // 16, _zero, 0)
    for t in range(ACCR // 16):
        denv[pl.ds(t * 16, 16)] = zero16f

    cnt = jnp.max(plsc.load_gather(cbuf, [jnp.zeros((16,), _i32)]))
    nch = (cnt + (K - 1)) // K
    mval = mbuf[...]

    def _chunk(ci, _):
        base = pl.multiple_of(wid * LCAP + ci * K, 8)
        pltpu.sync_copy(slist_ref.at[pl.ds(base, K)], slb)
        pltpu.sync_copy(dlist_ref.at[pl.ds(base, K)], dlb)
        pltpu.async_copy(xl_ref.at[slb], rows, sem).wait()
        for t in range(K // 16):
            sv = slb[pl.ds(t * 16, 16)]
            dv = dlb[pl.ds(t * 16, 16)]
            asg = plsc.load_gather(as_t, [sv])
            adg = plsc.load_gather(ad_t, [dv + lo])
            e = asg + adg
            e = jnp.where(e > 0.0, e, 0.2 * e)
            exb[pl.ds(t * 16, 16)] = jnp.exp(e - mval)

        def _edge(j, _):
            js = jnp.full((16,), j, _i32)
            dls = plsc.load_gather(dlb, [js])
            exs = plsc.load_gather(exb, [js])
            plsc.addupdate_scatter(denv, [dls], exs, mask=lane0)
            rbase = dls * H
            for jj in range(H // 16):
                rv = plsc.load_gather(rows, [js, jj * 16 + iot])
                plsc.addupdate_scatter(accv, [rbase + (jj * 16) + iot],
                                       rv * exs)
            return 0

        lax.fori_loop(0, K, _edge, 0)
        return 0

    lax.fori_loop(0, nch, _chunk, 0)

    pltpu.sync_copy(accv.at[pl.ds(0, RPT * H)],
                    acc_ref.at[pl.ds(pl.multiple_of(lo * H, 8), RPT * H)])
    pltpu.sync_copy(denv, dens_ref.at[pl.ds(pl.multiple_of(wid * ACCR, 8), ACCR)])


@functools.partial(
    pl.kernel,
    out_type=[
        jax.ShapeDtypeStruct((NPAD * H,), _f32),
        jax.ShapeDtypeStruct((NT * ACCR,), _f32),
    ],
    mesh=plsc.VectorSubcoreMesh(core_axis_name="c", subcore_axis_name="s"),
    compiler_params=pltpu.CompilerParams(needs_layout_passes=False),
    scratch_types=[
        pltpu.VMEM((TPAD,), _f32),
        pltpu.VMEM((TPAD,), _f32),
        pltpu.VMEM((16,), _f32),
        pltpu.VMEM((16,), _i32),
        pltpu.VMEM((K,), _i32),
        pltpu.VMEM((K,), _i32),
        pltpu.VMEM((K,), _f32),
        pltpu.VMEM((K, H), _f32),
        pltpu.VMEM((ACCR * H,), _f32),
        pltpu.VMEM((ACCR,), _f32),
        pltpu.SemaphoreType.DMA,
    ],
)
def _edge_pass(slist_ref, dlist_ref, counts_ref, asq_ref, adq_ref, xl_ref,
               mv_ref, acc_ref, dens_ref, *scr):
    _edge_body(slist_ref, dlist_ref, counts_ref, asq_ref, adq_ref, xl_ref,
               mv_ref, acc_ref, dens_ref, *scr)


def _pad_table(v):
    return jnp.pad(v.reshape(-1), (0, TPAD - N))


def _sc_layer(slist, dlist, counts, asr, adr, xl, m):
    asq = _pad_table(asr)
    adq = _pad_table(adr)
    mv = jnp.broadcast_to(m.reshape(()), (16,))
    accf, densf = _edge_pass(slist, dlist, counts, asq, adq, xl, mv)
    acc = accf.reshape(NPAD, H)[:N]
    den = densf.reshape(NT, ACCR)[:, :RPT].reshape(NPAD)[:N]
    den_b = jnp.broadcast_to(den[:, None], (N, H))
    return acc, den_b


def kernel(x, edge_index, W1, a_src1, a_dst1, b1, W2, a_src2, a_dst2, b2,
           Wp, bp):
    src = edge_index[0]
    dst = edge_index[1]
    slist, dlist, counts = _filter(src, dst)

    xl1, asr1, adr1, m1 = _tc1(x, W1, a_src1, a_dst1)
    acc1, den1b = _sc_layer(slist, dlist, counts, asr1, adr1, xl1, m1)

    xl2, asr2, adr2, m2 = _tc2(acc1, den1b, xl1, a_src1, a_dst1, m1, b1,
                               W2, a_src2, a_dst2)
    acc2, den2b = _sc_layer(slist, dlist, counts, asr2, adr2, xl2, m2)

    return _tc3(acc2, den2b, xl2, a_src2, a_dst2, m2, b2, Wp, bp)
